# baseline (speedup 1.0000x reference)
import jax
import jax.numpy as jnp
from jax.experimental import pallas as pl
from jax.experimental.pallas import tpu as pltpu

_BB = 256
_NS = 4
_AHEAD = 3


def _body(idx_ref, ppr_hbm, x_hbm, w_ref, b_ref, out_ref, enc, buf, xv, sems,
          xsem):
    i = pl.program_id(0)
    nsteps = pl.num_programs(0)
    slot = jax.lax.rem(i, _NS)

    def _issue(step, s):
        def one(k, carry):
            r = idx_ref[step * _BB + k]
            pltpu.make_async_copy(
                ppr_hbm.at[r], buf.at[s, k], sems.at[s]
            ).start()
            return carry

        jax.lax.fori_loop(0, _BB, one, 0, unroll=16)

    @pl.when(i == 0)
    def _():
        for s in range(_AHEAD + 1):
            _issue(s, s)
        xcopy = pltpu.make_async_copy(x_hbm, xv, xsem)
        xcopy.start()
        xcopy.wait()
        enc[...] = (
            jnp.dot(xv[...], w_ref[...], preferred_element_type=jnp.float32)
            + b_ref[...]
        ).astype(jnp.bfloat16)

    @pl.when(jnp.logical_and(i > 0, i + _AHEAD < nsteps))
    def _():
        _issue(i + _AHEAD, jax.lax.rem(i + _AHEAD, _NS))

    pltpu.make_async_copy(
        ppr_hbm.at[pl.ds(0, _BB)], buf.at[slot], sems.at[slot]
    ).wait()

    out_ref[...] = jnp.dot(
        buf[slot].astype(jnp.bfloat16),
        enc[...],
        preferred_element_type=jnp.float32,
    )


def kernel(X, idx, ppr, W, b):
    n, d = X.shape
    dout = W.shape[1]
    bsz = idx.shape[0]

    out = pl.pallas_call(
        _body,
        grid_spec=pltpu.PrefetchScalarGridSpec(
            num_scalar_prefetch=1,
            grid=(bsz // _BB,),
            in_specs=[
                pl.BlockSpec(memory_space=pltpu.HBM),
                pl.BlockSpec(memory_space=pltpu.HBM),
                pl.BlockSpec((d, dout), lambda i, idx_ref: (0, 0)),
                pl.BlockSpec((1, dout), lambda i, idx_ref: (0, 0)),
            ],
            out_specs=pl.BlockSpec((_BB, dout), lambda i, idx_ref: (i, 0)),
            scratch_shapes=[
                pltpu.VMEM((n, dout), jnp.bfloat16),
                pltpu.VMEM((_NS, _BB, n), jnp.float32),
                pltpu.VMEM((n, d), jnp.float32),
                pltpu.SemaphoreType.DMA((_NS,)),
                pltpu.SemaphoreType.DMA,
            ],
        ),
        out_shape=jax.ShapeDtypeStruct((bsz, dout), jnp.float32),
    )(idx.astype(jnp.int32), ppr, X, W, b.reshape(1, dout))
    return out


# speedup vs baseline: 1.1160x; 1.1160x over previous
import jax
import jax.numpy as jnp
from jax.experimental import pallas as pl
from jax.experimental.pallas import tpu as pltpu

_BB = 256
_NS = 4
_AHEAD = 2


def _body(idx_ref, ppr_hbm, x_ref, w_ref, b_ref, out_ref, enc, buf, sems):
    i = pl.program_id(0)
    nsteps = pl.num_programs(0)
    slot = jax.lax.rem(i, _NS)

    def _issue(step, s):
        def one(k, carry):
            r = idx_ref[step * _BB + k]
            pltpu.make_async_copy(
                ppr_hbm.at[r], buf.at[s, k], sems.at[s]
            ).start()
            return carry

        jax.lax.fori_loop(0, _BB, one, 0, unroll=16)

    @pl.when(i == 0)
    def _():
        for s in range(_AHEAD + 1):
            _issue(s, s)
        enc[...] = (
            jnp.dot(x_ref[...], w_ref[...], preferred_element_type=jnp.float32)
            + b_ref[...]
        ).astype(jnp.bfloat16)

    @pl.when(jnp.logical_and(i > 0, i + _AHEAD < nsteps))
    def _():
        _issue(i + _AHEAD, jax.lax.rem(i + _AHEAD, _NS))

    pltpu.make_async_copy(
        ppr_hbm.at[pl.ds(0, _BB)], buf.at[slot], sems.at[slot]
    ).wait()

    out_ref[...] = jnp.dot(
        buf[slot].astype(jnp.bfloat16),
        enc[...],
        preferred_element_type=jnp.float32,
    )


def kernel(X, idx, ppr, W, b):
    n, d = X.shape
    dout = W.shape[1]
    bsz = idx.shape[0]

    out = pl.pallas_call(
        _body,
        grid_spec=pltpu.PrefetchScalarGridSpec(
            num_scalar_prefetch=1,
            grid=(bsz // _BB,),
            in_specs=[
                pl.BlockSpec(memory_space=pltpu.HBM),
                pl.BlockSpec((n, d), lambda i, idx_ref: (0, 0)),
                pl.BlockSpec((d, dout), lambda i, idx_ref: (0, 0)),
                pl.BlockSpec((1, dout), lambda i, idx_ref: (0, 0)),
            ],
            out_specs=pl.BlockSpec((_BB, dout), lambda i, idx_ref: (i, 0)),
            scratch_shapes=[
                pltpu.VMEM((n, dout), jnp.bfloat16),
                pltpu.VMEM((_NS, _BB, n), jnp.float32),
                pltpu.SemaphoreType.DMA((_NS,)),
            ],
        ),
        out_shape=jax.ShapeDtypeStruct((bsz, dout), jnp.float32),
    )(idx.astype(jnp.int32), ppr, X, W, b.reshape(1, dout))
    return out
